# baseline (speedup 1.0000x reference)
import functools

import jax
import jax.numpy as jnp
from jax import lax
from jax.experimental import pallas as pl
from jax.experimental.pallas import tpu as pltpu
from jax.experimental.pallas import tpu_sc as plsc

_B = 16384
_F = 26
_V = 1000000
_D = 32
_BN_EPS = 1e-3
_N = _B * _F
_NSPLIT = 2
_CH = 832


@functools.cache
def _make_gather(n):
  info = plsc.get_sparse_core_info()
  nc, ns = info.num_cores, info.num_subcores
  nw = nc * ns
  per_w = n // nw
  n_ch = per_w // _CH
  assert per_w % _CH == 0

  mesh = plsc.VectorSubcoreMesh(core_axis_name="c", subcore_axis_name="s")

  @functools.partial(
      pl.kernel,
      mesh=mesh,
      compiler_params=pltpu.CompilerParams(use_tc_tiling_on_sc=False),
      out_type=(
          jax.ShapeDtypeStruct((n, _D), jnp.float32),
          jax.ShapeDtypeStruct((n,), jnp.float32),
      ),
      scratch_types=[
          pltpu.VMEM((_CH,), jnp.int32),
          pltpu.VMEM((_CH, 128), jnp.float32),
          pltpu.VMEM((_CH,), jnp.float32),
          pltpu.SemaphoreType.DMA,
          pltpu.SemaphoreType.DMA,
      ],
  )
  def gather(emb_hbm, fw_hbm, idx_hbm, emb_out, fw_out,
             idx_v, rows_v, fw_v, sem_e, sem_f):
    wid = lax.axis_index("s") * nc + lax.axis_index("c")

    def body(c, carry):
      base = wid * per_w + c * _CH
      pltpu.sync_copy(idx_hbm.at[pl.ds(base, _CH)], idx_v)
      cp_e = pltpu.async_copy(emb_hbm.at[idx_v], rows_v, sem_e)
      cp_f = pltpu.async_copy(fw_hbm.at[idx_v], fw_v, sem_f)
      cp_e.wait()
      cp_f.wait()
      pltpu.sync_copy(rows_v.at[:, :_D], emb_out.at[pl.ds(base, _CH)])
      pltpu.sync_copy(fw_v, fw_out.at[pl.ds(base, _CH)])
      return carry

    lax.fori_loop(0, n_ch, body, 0)

  return gather


def _mlp_body(emb_ref, fv_ref, fw_ref, e_ref, s_ref,
              w0_ref, b0_ref, w1_ref, b1_ref, w2_ref, b2_ref,
              wo1_ref, wo2_ref, wo3_ref, bo_ref, out_ref):
  fv = fv_ref[...]
  fve = jnp.dot(fv, e_ref[...], preferred_element_type=jnp.float32)
  ev = emb_ref[...] * fve
  s1 = jnp.dot(ev, s_ref[...], preferred_element_type=jnp.float32)
  s2 = jnp.dot(ev * ev, s_ref[...], preferred_element_type=jnp.float32)
  fm2 = 0.5 * (s1 * s1 - s2)
  d = jnp.maximum(
      jnp.dot(ev, w0_ref[...], preferred_element_type=jnp.float32)
      + b0_ref[...], 0.0)
  d = jnp.maximum(
      jnp.dot(d, w1_ref[...], preferred_element_type=jnp.float32)
      + b1_ref[...], 0.0)
  d = jnp.maximum(
      jnp.dot(d, w2_ref[...], preferred_element_type=jnp.float32)
      + b2_ref[...], 0.0)
  fm1 = fv * fw_ref[...]
  logit = (jnp.sum(fm1 * wo1_ref[...], axis=1, keepdims=True)
           + jnp.sum(fm2 * wo2_ref[...], axis=1, keepdims=True)
           + jnp.sum(d * wo3_ref[...], axis=1, keepdims=True)
           + bo_ref[...])
  out_ref[...] = 1.0 / (1.0 + jnp.exp(-logit))


def kernel(feature_index, feature_value, fw_table, emb_table,
           W0, b0, g0, be0, W1, b1, g1, be1, W2, b2, g2, be2,
           Wout, bout):
  s = 1.0 / jnp.sqrt(jnp.float32(1.0 + _BN_EPS))
  w0 = W0 * (g0 * s)[None, :]
  bb0 = (b0 * s * g0 + be0)[None, :]
  w1 = W1 * (g1 * s)[None, :]
  bb1 = (b1 * s * g1 + be1)[None, :]
  w2 = W2 * (g2 * s)[None, :]
  bb2 = (b2 * s * g2 + be2)[None, :]

  eye_f = jnp.eye(_F, dtype=jnp.float32)
  e_mat = jnp.kron(eye_f, jnp.ones((1, _D), jnp.float32))
  s_mat = jnp.tile(jnp.eye(_D, dtype=jnp.float32), (_F, 1))
  wo1 = Wout[:_F].reshape(1, _F)
  wo2 = Wout[_F:_F + _D].reshape(1, _D)
  wo3 = Wout[_F + _D:].reshape(1, _D)
  bo = bout.reshape(1, 1)

  r = 256
  fd = _F * _D
  bs = _B // _NSPLIT
  ns = bs * _F
  fw_flat = fw_table.reshape(-1)
  emb_lin = jnp.pad(emb_table, ((0, 0), (0, 128 - _D)))
  full = lambda shape: pl.BlockSpec(shape, lambda i: (0, 0))
  gather = _make_gather(ns)

  outs = []
  for k in range(_NSPLIT):
    fv_k = lax.slice_in_dim(feature_value, k * bs, (k + 1) * bs, axis=0)
    idx_k = lax.slice_in_dim(feature_index, k * bs, (k + 1) * bs,
                             axis=0).reshape(-1)
    emb_rows, fw_rows = gather(emb_lin, fw_flat, idx_k)
    emb_flat = emb_rows.reshape(bs, fd)
    fw = fw_rows.reshape(bs, _F)
    out_k = pl.pallas_call(
        _mlp_body,
        grid=(bs // r,),
        in_specs=[
            pl.BlockSpec((r, fd), lambda i: (i, 0)),
            pl.BlockSpec((r, _F), lambda i: (i, 0)),
            pl.BlockSpec((r, _F), lambda i: (i, 0)),
            full((_F, fd)),
            full((fd, _D)),
            full((fd, 128)),
            full((1, 128)),
            full((128, 64)),
            full((1, 64)),
            full((64, 32)),
            full((1, 32)),
            full((1, _F)),
            full((1, _D)),
            full((1, _D)),
            full((1, 1)),
        ],
        out_specs=pl.BlockSpec((r, 1), lambda i: (i, 0)),
        out_shape=jax.ShapeDtypeStruct((bs, 1), jnp.float32),
    )(emb_flat, fv_k, fw, e_mat, s_mat,
      w0, bb0, w1, bb1, w2, bb2, wo1, wo2, wo3, bo)
    outs.append(out_k)
  return jnp.concatenate(outs, axis=0)


# speedup vs baseline: 1.0767x; 1.0767x over previous
import functools

import jax
import jax.numpy as jnp
from jax import lax
from jax.experimental import pallas as pl
from jax.experimental.pallas import tpu as pltpu
from jax.experimental.pallas import tpu_sc as plsc

_B = 16384
_F = 26
_V = 1000000
_D = 32
_BN_EPS = 1e-3
_N = _B * _F
_NSPLIT = 2
_CH = 832


@functools.cache
def _make_gather(n):
  info = plsc.get_sparse_core_info()
  nc, ns = info.num_cores, info.num_subcores
  nw = nc * ns
  per_w = n // nw
  n_ch = per_w // _CH
  assert per_w % _CH == 0

  mesh = plsc.VectorSubcoreMesh(core_axis_name="c", subcore_axis_name="s")

  @functools.partial(
      pl.kernel,
      mesh=mesh,
      compiler_params=pltpu.CompilerParams(use_tc_tiling_on_sc=False),
      out_type=(
          jax.ShapeDtypeStruct((n, _D), jnp.float32),
          jax.ShapeDtypeStruct((n,), jnp.float32),
      ),
      scratch_types=[
          pltpu.VMEM((_CH,), jnp.int32),
          pltpu.VMEM((_CH, _D), jnp.float32),
          pltpu.VMEM((_CH,), jnp.float32),
          pltpu.SemaphoreType.DMA,
          pltpu.SemaphoreType.DMA,
      ],
  )
  def gather(emb_hbm, fw_hbm, idx_hbm, emb_out, fw_out,
             idx_v, rows_v, fw_v, sem_e, sem_f):
    wid = lax.axis_index("s") * nc + lax.axis_index("c")

    def body(c, carry):
      base = wid * per_w + c * _CH
      pltpu.sync_copy(idx_hbm.at[pl.ds(base, _CH)], idx_v)
      cp_e = pltpu.async_copy(emb_hbm.at[idx_v], rows_v, sem_e)
      cp_f = pltpu.async_copy(fw_hbm.at[idx_v], fw_v, sem_f)
      cp_e.wait()
      cp_f.wait()
      pltpu.sync_copy(rows_v, emb_out.at[pl.ds(base, _CH)])
      pltpu.sync_copy(fw_v, fw_out.at[pl.ds(base, _CH)])
      return carry

    lax.fori_loop(0, n_ch, body, 0)

  return gather


def _mlp_body(emb_ref, fv_ref, fw_ref, e_ref, s_ref,
              w0_ref, b0_ref, w1_ref, b1_ref, w2_ref, b2_ref,
              wo1_ref, wo2_ref, wo3_ref, bo_ref, out_ref):
  fv = fv_ref[...]
  fve = jnp.dot(fv, e_ref[...], preferred_element_type=jnp.float32)
  ev = emb_ref[...] * fve
  s1 = jnp.dot(ev, s_ref[...], preferred_element_type=jnp.float32)
  s2 = jnp.dot(ev * ev, s_ref[...], preferred_element_type=jnp.float32)
  fm2 = 0.5 * (s1 * s1 - s2)
  d = jnp.maximum(
      jnp.dot(ev, w0_ref[...], preferred_element_type=jnp.float32)
      + b0_ref[...], 0.0)
  d = jnp.maximum(
      jnp.dot(d, w1_ref[...], preferred_element_type=jnp.float32)
      + b1_ref[...], 0.0)
  d = jnp.maximum(
      jnp.dot(d, w2_ref[...], preferred_element_type=jnp.float32)
      + b2_ref[...], 0.0)
  fm1 = fv * fw_ref[...]
  logit = (jnp.sum(fm1 * wo1_ref[...], axis=1, keepdims=True)
           + jnp.sum(fm2 * wo2_ref[...], axis=1, keepdims=True)
           + jnp.sum(d * wo3_ref[...], axis=1, keepdims=True)
           + bo_ref[...])
  out_ref[...] = 1.0 / (1.0 + jnp.exp(-logit))


def kernel(feature_index, feature_value, fw_table, emb_table,
           W0, b0, g0, be0, W1, b1, g1, be1, W2, b2, g2, be2,
           Wout, bout):
  s = 1.0 / jnp.sqrt(jnp.float32(1.0 + _BN_EPS))
  w0 = W0 * (g0 * s)[None, :]
  bb0 = (b0 * s * g0 + be0)[None, :]
  w1 = W1 * (g1 * s)[None, :]
  bb1 = (b1 * s * g1 + be1)[None, :]
  w2 = W2 * (g2 * s)[None, :]
  bb2 = (b2 * s * g2 + be2)[None, :]

  eye_f = jnp.eye(_F, dtype=jnp.float32)
  e_mat = jnp.kron(eye_f, jnp.ones((1, _D), jnp.float32))
  s_mat = jnp.tile(jnp.eye(_D, dtype=jnp.float32), (_F, 1))
  wo1 = Wout[:_F].reshape(1, _F)
  wo2 = Wout[_F:_F + _D].reshape(1, _D)
  wo3 = Wout[_F + _D:].reshape(1, _D)
  bo = bout.reshape(1, 1)

  r = 256
  fd = _F * _D
  bs = _B // _NSPLIT
  ns = bs * _F
  fw_flat = fw_table.reshape(-1)
  full = lambda shape: pl.BlockSpec(shape, lambda i: (0, 0))
  gather = _make_gather(ns)

  outs = []
  for k in range(_NSPLIT):
    fv_k = lax.slice_in_dim(feature_value, k * bs, (k + 1) * bs, axis=0)
    idx_k = lax.slice_in_dim(feature_index, k * bs, (k + 1) * bs,
                             axis=0).reshape(-1)
    emb_rows, fw_rows = gather(emb_table, fw_flat, idx_k)
    emb_flat = emb_rows.reshape(bs, fd)
    fw = fw_rows.reshape(bs, _F)
    out_k = pl.pallas_call(
        _mlp_body,
        grid=(bs // r,),
        in_specs=[
            pl.BlockSpec((r, fd), lambda i: (i, 0)),
            pl.BlockSpec((r, _F), lambda i: (i, 0)),
            pl.BlockSpec((r, _F), lambda i: (i, 0)),
            full((_F, fd)),
            full((fd, _D)),
            full((fd, 128)),
            full((1, 128)),
            full((128, 64)),
            full((1, 64)),
            full((64, 32)),
            full((1, 32)),
            full((1, _F)),
            full((1, _D)),
            full((1, _D)),
            full((1, 1)),
        ],
        out_specs=pl.BlockSpec((r, 1), lambda i: (i, 0)),
        out_shape=jax.ShapeDtypeStruct((bs, 1), jnp.float32),
    )(emb_flat, fv_k, fw, e_mat, s_mat,
      w0, bb0, w1, bb1, w2, bb2, wo1, wo2, wo3, bo)
    outs.append(out_k)
  return jnp.concatenate(outs, axis=0)
